# baseline (speedup 1.0000x reference)
import functools

import jax
import jax.numpy as jnp
from jax import lax
from jax.experimental import pallas as pl
from jax.experimental.pallas import tpu as pltpu
from jax.experimental.pallas import tpu_sc as plsc

N = 10000
E = 320000
D = 128

NC = 2
NS = 16
NW = NC * NS
CHUNK = 128
CPT = 80
EPT = CPT * CHUNK
E_PAD = NW * EPT
N_PAD = 10240
RPT = N_PAD // NS
ZR = 40

_MESH = plsc.VectorSubcoreMesh(
    core_axis_name="c", subcore_axis_name="s", num_cores=NC, num_subcores=NS
)


@functools.partial(
    pl.kernel,
    out_type=jax.ShapeDtypeStruct((NC, N_PAD), jnp.float32),
    mesh=_MESH,
    scratch_types=[
        pltpu.VMEM((CPT, CHUNK), jnp.int32),
        pltpu.VMEM((CHUNK,), jnp.float32),
        pltpu.VMEM((RPT,), jnp.float32),
        pltpu.VMEM_SHARED((N_PAD,), jnp.float32),
    ],
)
def _deg_kernel(dst_hbm, out_hbm, didx, ones, zrow, deg):
    c = lax.axis_index("c")
    s = lax.axis_index("s")
    wid = s * NC + c

    def fill_ones(k, _):
        ones[pl.ds(k * 16, 16)] = jnp.full((16,), 1.0, jnp.float32)
        return _

    lax.fori_loop(0, CHUNK // 16, fill_ones, None)

    def fill_zeros(k, _):
        zrow[pl.ds(k * 16, 16)] = jnp.zeros((16,), jnp.float32)
        return _

    lax.fori_loop(0, RPT // 16, fill_zeros, None)

    pltpu.sync_copy(dst_hbm.at[wid], didx)
    pltpu.sync_copy(zrow, deg.at[pl.ds(s * RPT, RPT)])
    plsc.subcore_barrier()

    def chunk(ci, _):
        pltpu.sync_copy(ones, deg.at[didx.at[ci]], add=True)
        return _

    lax.fori_loop(0, CPT, chunk, None)
    plsc.subcore_barrier()
    pltpu.sync_copy(deg.at[pl.ds(s * RPT, RPT)], out_hbm.at[c, pl.ds(s * RPT, RPT)])


@functools.partial(
    pl.kernel,
    out_type=jax.ShapeDtypeStruct((NC, N_PAD, D), jnp.float32),
    mesh=_MESH,
    scratch_types=[
        pltpu.VMEM((CPT, CHUNK), jnp.int32),
        pltpu.VMEM((CPT, CHUNK), jnp.int32),
        pltpu.VMEM((CHUNK, D), jnp.float32),
        pltpu.VMEM_SHARED((N_PAD, D), jnp.float32),
        pltpu.SemaphoreType.DMA,
    ],
)
def _hop_kernel(g_hbm, src_hbm, dst_hbm, zeros_hbm, out_hbm, sidx, didx, rows,
                acc, sem):
    c = lax.axis_index("c")
    s = lax.axis_index("s")
    wid = s * NC + c

    pltpu.sync_copy(zeros_hbm, acc.at[pl.ds(s * RPT, RPT)])
    plsc.subcore_barrier()

    pltpu.sync_copy(src_hbm.at[wid], sidx)
    pltpu.sync_copy(dst_hbm.at[wid], didx)

    def chunk(ci, _):
        pltpu.async_copy(g_hbm.at[sidx.at[ci]], rows, sem).wait()
        pltpu.sync_copy(rows, acc.at[didx.at[ci]], add=True)
        return _

    lax.fori_loop(0, CPT, chunk, None)
    plsc.subcore_barrier()

    sl = pl.ds(s * RPT, RPT)
    pltpu.sync_copy(acc.at[sl], out_hbm.at[c, sl])


_ROWS = 80
_GRID = N // _ROWS


def _scale0_body(deg_ref, x_ref, o_ref):
    dsum = deg_ref[0] + deg_ref[1] + 1.0
    o_ref[...] = lax.rsqrt(dsum) * x_ref[...]


def _scale_mid_body(deg_ref, parts_ref, g_ref, o_ref):
    dsum = deg_ref[0] + deg_ref[1] + 1.0
    o_ref[...] = (parts_ref[0] + parts_ref[1] + g_ref[...]) / dsum


def _final_body(deg_ref, parts_ref, g_ref, w_ref, o_ref):
    dsum = deg_ref[0] + deg_ref[1] + 1.0
    h = lax.rsqrt(dsum) * (parts_ref[0] + parts_ref[1] + g_ref[...])
    o_ref[...] = jnp.dot(h, w_ref[...], preferred_element_type=jnp.float32)


_deg_spec = pl.BlockSpec((NC, _ROWS, 1), lambda i: (0, i, 0))
_row_spec = pl.BlockSpec((_ROWS, D), lambda i: (i, 0))
_parts_spec = pl.BlockSpec((NC, _ROWS, D), lambda i: (0, i, 0))
_w_spec = pl.BlockSpec((D, D), lambda i: (0, 0))
_out_shape = jax.ShapeDtypeStruct((N, D), jnp.float32)

_scale0 = pl.pallas_call(
    _scale0_body, grid=(_GRID,),
    in_specs=[_deg_spec, _row_spec], out_specs=_row_spec, out_shape=_out_shape,
)
_scale_mid = pl.pallas_call(
    _scale_mid_body, grid=(_GRID,),
    in_specs=[_deg_spec, _parts_spec, _row_spec],
    out_specs=_row_spec, out_shape=_out_shape,
)
_final = pl.pallas_call(
    _final_body, grid=(_GRID,),
    in_specs=[_deg_spec, _parts_spec, _row_spec, _w_spec],
    out_specs=_row_spec, out_shape=_out_shape,
)


@jax.jit
def kernel(x, edge_index, W):
    src = edge_index[0].astype(jnp.int32)
    dst = edge_index[1].astype(jnp.int32)
    pad = E_PAD - E
    src3 = jnp.concatenate([src, jnp.zeros((pad,), jnp.int32)]).reshape(NW, CPT, CHUNK)
    dst3 = jnp.concatenate([dst, jnp.full((pad,), N, jnp.int32)]).reshape(NW, CPT, CHUNK)

    zblk = jnp.zeros((RPT, D), jnp.float32)

    deg_parts = _deg_kernel(dst3).reshape(NC, N_PAD, 1)
    g0 = _scale0(deg_parts, x)
    p1 = _hop_kernel(g0, src3, dst3, zblk)
    g1 = _scale_mid(deg_parts, p1, g0)
    p2 = _hop_kernel(g1, src3, dst3, zblk)
    return _final(deg_parts, p2, g1, W)


# speedup vs baseline: 1.0266x; 1.0266x over previous
import functools

import jax
import jax.numpy as jnp
from jax import lax
from jax.experimental import pallas as pl
from jax.experimental.pallas import tpu as pltpu
from jax.experimental.pallas import tpu_sc as plsc

N = 10000
E = 320000
D = 128

NC = 2
NS = 16
NW = NC * NS
CHUNK = 128
CPT = 80
EPT = CPT * CHUNK
E_PAD = NW * EPT
N_PAD = 10240
RPT = N_PAD // NS
ZR = 40

_MESH = plsc.VectorSubcoreMesh(
    core_axis_name="c", subcore_axis_name="s", num_cores=NC, num_subcores=NS
)


@functools.partial(
    pl.kernel,
    out_type=jax.ShapeDtypeStruct((NC, N_PAD), jnp.float32),
    mesh=_MESH,
    scratch_types=[
        pltpu.VMEM((CPT, CHUNK), jnp.int32),
        pltpu.VMEM((CHUNK,), jnp.float32),
        pltpu.VMEM((RPT,), jnp.float32),
        pltpu.VMEM_SHARED((N_PAD,), jnp.float32),
    ],
)
def _deg_kernel(dst_hbm, out_hbm, didx, ones, zrow, deg):
    c = lax.axis_index("c")
    s = lax.axis_index("s")
    wid = s * NC + c

    def fill_ones(k, _):
        ones[pl.ds(k * 16, 16)] = jnp.full((16,), 1.0, jnp.float32)
        return _

    lax.fori_loop(0, CHUNK // 16, fill_ones, None)

    def fill_zeros(k, _):
        zrow[pl.ds(k * 16, 16)] = jnp.zeros((16,), jnp.float32)
        return _

    lax.fori_loop(0, RPT // 16, fill_zeros, None)

    pltpu.sync_copy(dst_hbm.at[wid], didx)
    pltpu.sync_copy(zrow, deg.at[pl.ds(s * RPT, RPT)])
    plsc.subcore_barrier()

    def chunk(ci, _):
        pltpu.sync_copy(ones, deg.at[didx.at[ci]], add=True)
        return _

    lax.fori_loop(0, CPT, chunk, None)
    plsc.subcore_barrier()
    pltpu.sync_copy(deg.at[pl.ds(s * RPT, RPT)], out_hbm.at[c, pl.ds(s * RPT, RPT)])


@functools.partial(
    pl.kernel,
    out_type=jax.ShapeDtypeStruct((NC, N_PAD, D), jnp.float32),
    mesh=_MESH,
    scratch_types=[
        pltpu.VMEM((CPT, CHUNK), jnp.int32),
        pltpu.VMEM((CPT, CHUNK), jnp.int32),
        pltpu.VMEM((CHUNK, D), jnp.float32),
        pltpu.VMEM((ZR, D), jnp.float32),
        pltpu.VMEM_SHARED((N_PAD, D), jnp.float32),
        pltpu.SemaphoreType.DMA,
    ],
)
def _hop_kernel(g_hbm, src_hbm, dst_hbm, out_hbm, sidx, didx, rows, zbuf,
                acc, sem):
    c = lax.axis_index("c")
    s = lax.axis_index("s")
    wid = s * NC + c

    def fill_zeros(k, _):
        zbuf[k // 8, pl.ds((k % 8) * 16, 16)] = jnp.zeros((16,), jnp.float32)
        return _

    lax.fori_loop(0, ZR * (D // 16), fill_zeros, None)

    def zero_acc(k, _):
        pltpu.sync_copy(zbuf, acc.at[pl.ds(s * RPT + k * ZR, ZR)])
        return _

    lax.fori_loop(0, RPT // ZR, zero_acc, None)
    plsc.subcore_barrier()

    pltpu.sync_copy(src_hbm.at[wid], sidx)
    pltpu.sync_copy(dst_hbm.at[wid], didx)

    def chunk(ci, _):
        pltpu.async_copy(g_hbm.at[sidx.at[ci]], rows, sem).wait()
        pltpu.sync_copy(rows, acc.at[didx.at[ci]], add=True)
        return _

    lax.fori_loop(0, CPT, chunk, None)
    plsc.subcore_barrier()

    def flush(k, _):
        sl = pl.ds(s * RPT + k * ZR, ZR)
        pltpu.sync_copy(acc.at[sl], out_hbm.at[c, sl])
        return _

    lax.fori_loop(0, RPT // ZR, flush, None)


_ROWS = 80
_GRID = N // _ROWS


def _scale0_body(deg_ref, x_ref, o_ref):
    dsum = deg_ref[0] + deg_ref[1] + 1.0
    o_ref[...] = lax.rsqrt(dsum) * x_ref[...]


def _scale_mid_body(deg_ref, parts_ref, g_ref, o_ref):
    dsum = deg_ref[0] + deg_ref[1] + 1.0
    o_ref[...] = (parts_ref[0] + parts_ref[1] + g_ref[...]) / dsum


def _final_body(deg_ref, parts_ref, g_ref, w_ref, o_ref):
    dsum = deg_ref[0] + deg_ref[1] + 1.0
    h = lax.rsqrt(dsum) * (parts_ref[0] + parts_ref[1] + g_ref[...])
    o_ref[...] = jnp.dot(h, w_ref[...], preferred_element_type=jnp.float32)


_deg_spec = pl.BlockSpec((NC, _ROWS, 1), lambda i: (0, i, 0))
_row_spec = pl.BlockSpec((_ROWS, D), lambda i: (i, 0))
_parts_spec = pl.BlockSpec((NC, _ROWS, D), lambda i: (0, i, 0))
_w_spec = pl.BlockSpec((D, D), lambda i: (0, 0))
_out_shape = jax.ShapeDtypeStruct((N, D), jnp.float32)

_scale0 = pl.pallas_call(
    _scale0_body, grid=(_GRID,),
    in_specs=[_deg_spec, _row_spec], out_specs=_row_spec, out_shape=_out_shape,
)
_scale_mid = pl.pallas_call(
    _scale_mid_body, grid=(_GRID,),
    in_specs=[_deg_spec, _parts_spec, _row_spec],
    out_specs=_row_spec, out_shape=_out_shape,
)
_final = pl.pallas_call(
    _final_body, grid=(_GRID,),
    in_specs=[_deg_spec, _parts_spec, _row_spec, _w_spec],
    out_specs=_row_spec, out_shape=_out_shape,
)


@jax.jit
def kernel(x, edge_index, W):
    src = edge_index[0].astype(jnp.int32)
    dst = edge_index[1].astype(jnp.int32)
    pad = E_PAD - E
    src3 = jnp.concatenate([src, jnp.zeros((pad,), jnp.int32)]).reshape(NW, CPT, CHUNK)
    dst3 = jnp.concatenate([dst, jnp.full((pad,), N, jnp.int32)]).reshape(NW, CPT, CHUNK)

    deg_parts = _deg_kernel(dst3).reshape(NC, N_PAD, 1)
    g0 = _scale0(deg_parts, x)
    p1 = _hop_kernel(g0, src3, dst3)
    g1 = _scale_mid(deg_parts, p1, g0)
    p2 = _hop_kernel(g1, src3, dst3)
    return _final(deg_parts, p2, g1, W)


# speedup vs baseline: 2.1746x; 2.1183x over previous
import functools

import jax
import jax.numpy as jnp
from jax import lax
from jax.experimental import pallas as pl
from jax.experimental.pallas import tpu as pltpu
from jax.experimental.pallas import tpu_sc as plsc

N = 10000
E = 320000
D = 128

NC = 2
NS = 16
NW = NC * NS
CHUNK = 128
CPT = 80
EPT = CPT * CHUNK
E_PAD = NW * EPT
N_PAD = 10240
RPT = N_PAD // NS
ZR = 40

_MESH = plsc.VectorSubcoreMesh(
    core_axis_name="c", subcore_axis_name="s", num_cores=NC, num_subcores=NS
)


@functools.partial(
    pl.kernel,
    out_type=jax.ShapeDtypeStruct((NC, N_PAD), jnp.float32),
    mesh=_MESH,
    scratch_types=[
        pltpu.VMEM((CPT, CHUNK), jnp.int32),
        pltpu.VMEM((CHUNK,), jnp.float32),
        pltpu.VMEM((RPT,), jnp.float32),
        pltpu.VMEM_SHARED((N_PAD,), jnp.float32),
    ],
)
def _deg_kernel(dst_hbm, out_hbm, didx, ones, zrow, deg):
    c = lax.axis_index("c")
    s = lax.axis_index("s")
    wid = s * NC + c

    def fill_ones(k, _):
        ones[pl.ds(k * 16, 16)] = jnp.full((16,), 1.0, jnp.float32)
        return _

    lax.fori_loop(0, CHUNK // 16, fill_ones, None)

    def fill_zeros(k, _):
        zrow[pl.ds(k * 16, 16)] = jnp.zeros((16,), jnp.float32)
        return _

    lax.fori_loop(0, RPT // 16, fill_zeros, None)

    pltpu.sync_copy(dst_hbm.at[wid], didx)
    pltpu.sync_copy(zrow, deg.at[pl.ds(s * RPT, RPT)])
    plsc.subcore_barrier()

    def chunk(ci, _):
        pltpu.sync_copy(ones, deg.at[didx.at[ci]], add=True)
        return _

    lax.fori_loop(0, CPT, chunk, None)
    plsc.subcore_barrier()
    pltpu.sync_copy(deg.at[pl.ds(s * RPT, RPT)], out_hbm.at[c, pl.ds(s * RPT, RPT)])


@functools.partial(
    pl.kernel,
    out_type=jax.ShapeDtypeStruct((NC, N_PAD, D), jnp.float32),
    mesh=_MESH,
    scratch_types=[
        pltpu.VMEM((CPT, CHUNK), jnp.int32),
        pltpu.VMEM((CPT, CHUNK), jnp.int32),
        pltpu.VMEM((CHUNK, D), jnp.float32),
        pltpu.VMEM((ZR, D), jnp.float32),
        pltpu.VMEM_SHARED((N_PAD, D), jnp.float32),
        pltpu.SemaphoreType.DMA,
    ],
)
def _hop_kernel(g_hbm, src_hbm, dst_hbm, out_hbm, sidx, didx, rows, zbuf,
                acc, sem):
    c = lax.axis_index("c")
    s = lax.axis_index("s")
    wid = s * NC + c

    def fill_zeros(k, _):
        zbuf[k // 8, pl.ds((k % 8) * 16, 16)] = jnp.zeros((16,), jnp.float32)
        return _

    lax.fori_loop(0, ZR * (D // 16), fill_zeros, None)

    def zero_acc(k, _):
        pltpu.sync_copy(zbuf, acc.at[pl.ds(s * RPT + k * ZR, ZR)])
        return _

    lax.fori_loop(0, RPT // ZR, zero_acc, None)
    plsc.subcore_barrier()

    pltpu.sync_copy(src_hbm.at[wid], sidx)
    pltpu.sync_copy(dst_hbm.at[wid], didx)

    def chunk(ci, _):
        pltpu.async_copy(g_hbm.at[sidx.at[ci]], rows, sem).wait()
        pltpu.sync_copy(rows, acc.at[didx.at[ci]], add=True)
        return _

    lax.fori_loop(0, CPT, chunk, None)
    plsc.subcore_barrier()

    def flush(k, _):
        sl = pl.ds(s * RPT + k * ZR, ZR)
        pltpu.sync_copy(acc.at[sl], out_hbm.at[c, sl])
        return _

    lax.fori_loop(0, RPT // ZR, flush, None)


_ROWS = 80
_GRID = N // _ROWS


def _scale0_body(deg_ref, x_ref, o_ref):
    dsum = deg_ref[0] + deg_ref[1] + 1.0
    o_ref[...] = lax.rsqrt(dsum) * x_ref[...]


def _scale_mid_body(deg_ref, parts_ref, g_ref, o_ref):
    dsum = deg_ref[0] + deg_ref[1] + 1.0
    o_ref[...] = (parts_ref[0] + parts_ref[1] + g_ref[...]) / dsum


def _final_body(deg_ref, parts_ref, g_ref, w_ref, o_ref):
    dsum = deg_ref[0] + deg_ref[1] + 1.0
    h = lax.rsqrt(dsum) * (parts_ref[0] + parts_ref[1] + g_ref[...])
    o_ref[...] = jnp.dot(h, w_ref[...], preferred_element_type=jnp.float32)


_deg_spec = pl.BlockSpec((NC, _ROWS, 1), lambda i: (0, i, 0))
_row_spec = pl.BlockSpec((_ROWS, D), lambda i: (i, 0))
_parts_spec = pl.BlockSpec((NC, _ROWS, D), lambda i: (0, i, 0))
_w_spec = pl.BlockSpec((D, D), lambda i: (0, 0))
_out_shape = jax.ShapeDtypeStruct((N, D), jnp.float32)

_scale0 = pl.pallas_call(
    _scale0_body, grid=(_GRID,),
    in_specs=[_deg_spec, _row_spec], out_specs=_row_spec, out_shape=_out_shape,
)
_scale_mid = pl.pallas_call(
    _scale_mid_body, grid=(_GRID,),
    in_specs=[_deg_spec, _parts_spec, _row_spec],
    out_specs=_row_spec, out_shape=_out_shape,
)
_final = pl.pallas_call(
    _final_body, grid=(_GRID,),
    in_specs=[_deg_spec, _parts_spec, _row_spec, _w_spec],
    out_specs=_row_spec, out_shape=_out_shape,
)


@jax.jit
def kernel(x, edge_index, W):
    src = edge_index[0].astype(jnp.int32)
    dst = edge_index[1].astype(jnp.int32)
    pad = E_PAD - E
    ar = jnp.arange(pad, dtype=jnp.int32)
    src_pad = ar % N
    dst_pad = N + 1 + ar % (N_PAD - N - 1)
    src3 = jnp.concatenate([src, src_pad]).reshape(NW, CPT, CHUNK)
    dst3 = jnp.concatenate([dst, dst_pad]).reshape(NW, CPT, CHUNK)

    deg_parts = _deg_kernel(dst3).reshape(NC, N_PAD, 1)
    g0 = _scale0(deg_parts, x)
    p1 = _hop_kernel(g0, src3, dst3)
    g1 = _scale_mid(deg_parts, p1, g0)
    p2 = _hop_kernel(g1, src3, dst3)
    return _final(deg_parts, p2, g1, W)


# speedup vs baseline: 2.7637x; 1.2709x over previous
import functools

import jax
import jax.numpy as jnp
from jax import lax
from jax.experimental import pallas as pl
from jax.experimental.pallas import tpu as pltpu
from jax.experimental.pallas import tpu_sc as plsc

N = 10000
E = 320000
D = 128

NC = 2
NS = 16
NW = NC * NS
CHUNK = 128
CPT = 80
PH = 40
EPT = CPT * CHUNK
E_PAD = NW * EPT
N_PAD = 10240
RPT = N_PAD // NS
ZR = 40

_MESH = plsc.VectorSubcoreMesh(
    core_axis_name="c", subcore_axis_name="s", num_cores=NC, num_subcores=NS
)


@functools.partial(
    pl.kernel,
    out_type=jax.ShapeDtypeStruct((NC, N_PAD), jnp.float32),
    mesh=_MESH,
    scratch_types=[
        pltpu.VMEM((CPT, CHUNK), jnp.int32),
        pltpu.VMEM((CHUNK,), jnp.float32),
        pltpu.VMEM((RPT,), jnp.float32),
        pltpu.VMEM_SHARED((N_PAD,), jnp.float32),
    ],
)
def _deg_kernel(dst_hbm, out_hbm, didx, ones, zrow, deg):
    c = lax.axis_index("c")
    s = lax.axis_index("s")
    wid = s * NC + c

    def fill_ones(k, _):
        ones[pl.ds(k * 16, 16)] = jnp.full((16,), 1.0, jnp.float32)
        return _

    lax.fori_loop(0, CHUNK // 16, fill_ones, None)

    def fill_zeros(k, _):
        zrow[pl.ds(k * 16, 16)] = jnp.zeros((16,), jnp.float32)
        return _

    lax.fori_loop(0, RPT // 16, fill_zeros, None)

    pltpu.sync_copy(dst_hbm.at[wid], didx)
    pltpu.sync_copy(zrow, deg.at[pl.ds(s * RPT, RPT)])
    plsc.subcore_barrier()

    def chunk(ci, _):
        pltpu.sync_copy(ones, deg.at[didx.at[ci]], add=True)
        return _

    lax.fori_loop(0, CPT, chunk, None)
    plsc.subcore_barrier()
    pltpu.sync_copy(deg.at[pl.ds(s * RPT, RPT)], out_hbm.at[c, pl.ds(s * RPT, RPT)])


@functools.partial(
    pl.kernel,
    out_type=jax.ShapeDtypeStruct((NC, N_PAD, D), jnp.float32),
    mesh=_MESH,
    scratch_types=[
        pltpu.VMEM((PH, CHUNK), jnp.int32),
        pltpu.VMEM((PH, CHUNK), jnp.int32),
        pltpu.VMEM((2, CHUNK, D), jnp.float32),
        pltpu.VMEM((ZR, D), jnp.float32),
        pltpu.VMEM_SHARED((N_PAD, D), jnp.float32),
        pltpu.SemaphoreType.DMA,
        pltpu.SemaphoreType.DMA,
    ],
)
def _hop_kernel(g_hbm, src_hbm, dst_hbm, out_hbm, sidx, didx, rows, zbuf,
                acc, gs0, gs1):
    gsem = (gs0, gs1)
    c = lax.axis_index("c")
    s = lax.axis_index("s")
    wid = s * NC + c

    def fill_zeros(k, _):
        zbuf[k // 8, pl.ds((k % 8) * 16, 16)] = jnp.zeros((16,), jnp.float32)
        return _

    lax.fori_loop(0, ZR * (D // 16), fill_zeros, None)

    def zero_acc(k, _):
        pltpu.sync_copy(zbuf, acc.at[pl.ds(s * RPT + k * ZR, ZR)])
        return _

    lax.fori_loop(0, RPT // ZR, zero_acc, None)
    plsc.subcore_barrier()

    for p in range(CPT // PH):
        pltpu.sync_copy(src_hbm.at[wid, pl.ds(p * PH, PH)], sidx)
        pltpu.sync_copy(dst_hbm.at[wid, pl.ds(p * PH, PH)], didx)
        pltpu.async_copy(g_hbm.at[sidx.at[0]], rows.at[0], gs0)
        pltpu.async_copy(g_hbm.at[sidx.at[1]], rows.at[1], gs1)

        def pair(j, _):
            for bb in range(2):
                k = j * 2 + bb
                pltpu.make_async_copy(
                    g_hbm.at[sidx.at[k]], rows.at[bb], gsem[bb]
                ).wait()
                pltpu.sync_copy(rows.at[bb], acc.at[didx.at[k]], add=True)
                pltpu.async_copy(g_hbm.at[sidx.at[k + 2]], rows.at[bb], gsem[bb])
            return _

        lax.fori_loop(0, PH // 2 - 1, pair, None)

        for k in (PH - 2, PH - 1):
            bb = k % 2
            pltpu.make_async_copy(
                g_hbm.at[sidx.at[k]], rows.at[bb], gsem[bb]
            ).wait()
            pltpu.sync_copy(rows.at[bb], acc.at[didx.at[k]], add=True)

    plsc.subcore_barrier()

    def flush(k, _):
        sl = pl.ds(s * RPT + k * ZR, ZR)
        pltpu.sync_copy(acc.at[sl], out_hbm.at[c, sl])
        return _

    lax.fori_loop(0, RPT // ZR, flush, None)


_ROWS = 80
_GRID = N // _ROWS


def _scale0_body(deg_ref, x_ref, o_ref):
    dsum = deg_ref[0] + deg_ref[1] + 1.0
    o_ref[...] = lax.rsqrt(dsum) * x_ref[...]


def _scale_mid_body(deg_ref, parts_ref, g_ref, o_ref):
    dsum = deg_ref[0] + deg_ref[1] + 1.0
    o_ref[...] = (parts_ref[0] + parts_ref[1] + g_ref[...]) / dsum


def _final_body(deg_ref, parts_ref, g_ref, w_ref, o_ref):
    dsum = deg_ref[0] + deg_ref[1] + 1.0
    h = lax.rsqrt(dsum) * (parts_ref[0] + parts_ref[1] + g_ref[...])
    o_ref[...] = jnp.dot(h, w_ref[...], preferred_element_type=jnp.float32)


_deg_spec = pl.BlockSpec((NC, _ROWS, 1), lambda i: (0, i, 0))
_row_spec = pl.BlockSpec((_ROWS, D), lambda i: (i, 0))
_parts_spec = pl.BlockSpec((NC, _ROWS, D), lambda i: (0, i, 0))
_w_spec = pl.BlockSpec((D, D), lambda i: (0, 0))
_out_shape = jax.ShapeDtypeStruct((N, D), jnp.float32)

_scale0 = pl.pallas_call(
    _scale0_body, grid=(_GRID,),
    in_specs=[_deg_spec, _row_spec], out_specs=_row_spec, out_shape=_out_shape,
)
_scale_mid = pl.pallas_call(
    _scale_mid_body, grid=(_GRID,),
    in_specs=[_deg_spec, _parts_spec, _row_spec],
    out_specs=_row_spec, out_shape=_out_shape,
)
_final = pl.pallas_call(
    _final_body, grid=(_GRID,),
    in_specs=[_deg_spec, _parts_spec, _row_spec, _w_spec],
    out_specs=_row_spec, out_shape=_out_shape,
)


@jax.jit
def kernel(x, edge_index, W):
    src = edge_index[0].astype(jnp.int32)
    dst = edge_index[1].astype(jnp.int32)
    pad = E_PAD - E
    ar = jnp.arange(pad, dtype=jnp.int32)
    src_pad = ar % N
    dst_pad = N + 1 + ar % (N_PAD - N - 1)
    src3 = jnp.concatenate([src, src_pad]).reshape(NW, CPT, CHUNK)
    dst3 = jnp.concatenate([dst, dst_pad]).reshape(NW, CPT, CHUNK)

    deg_parts = _deg_kernel(dst3).reshape(NC, N_PAD, 1)
    g0 = _scale0(deg_parts, x)
    p1 = _hop_kernel(g0, src3, dst3)
    g1 = _scale_mid(deg_parts, p1, g0)
    p2 = _hop_kernel(g1, src3, dst3)
    return _final(deg_parts, p2, g1, W)


# speedup vs baseline: 4.3377x; 1.5695x over previous
import functools

import jax
import jax.numpy as jnp
from jax import lax
from jax.experimental import pallas as pl
from jax.experimental.pallas import tpu as pltpu
from jax.experimental.pallas import tpu_sc as plsc

N = 10000
E = 320000
D = 128

NC = 2
NS = 16
NW = NC * NS
CHUNK = 128
CPT = 80
PH = 40
EPT = CPT * CHUNK
E_PAD = NW * EPT
N_PAD = 10240
RPT = N_PAD // NS
ZR = 40

_MESH = plsc.VectorSubcoreMesh(
    core_axis_name="c", subcore_axis_name="s", num_cores=NC, num_subcores=NS
)


@functools.partial(
    pl.kernel,
    out_type=jax.ShapeDtypeStruct((NC, N_PAD), jnp.float32),
    mesh=_MESH,
    scratch_types=[
        pltpu.VMEM((CPT, CHUNK), jnp.int32),
        pltpu.VMEM((CHUNK,), jnp.float32),
        pltpu.VMEM((RPT,), jnp.float32),
        pltpu.VMEM_SHARED((N_PAD,), jnp.float32),
    ],
)
def _deg_kernel(dst_hbm, out_hbm, didx, ones, zrow, deg):
    c = lax.axis_index("c")
    s = lax.axis_index("s")
    wid = s * NC + c

    def fill_ones(k, _):
        ones[pl.ds(k * 16, 16)] = jnp.full((16,), 1.0, jnp.float32)
        return _

    lax.fori_loop(0, CHUNK // 16, fill_ones, None)

    def fill_zeros(k, _):
        zrow[pl.ds(k * 16, 16)] = jnp.zeros((16,), jnp.float32)
        return _

    lax.fori_loop(0, RPT // 16, fill_zeros, None)

    pltpu.sync_copy(dst_hbm.at[wid], didx)
    pltpu.sync_copy(zrow, deg.at[pl.ds(s * RPT, RPT)])
    plsc.subcore_barrier()

    def chunk(ci, _):
        pltpu.sync_copy(ones, deg.at[didx.at[ci]], add=True)
        return _

    lax.fori_loop(0, CPT, chunk, None)
    plsc.subcore_barrier()
    pltpu.sync_copy(deg.at[pl.ds(s * RPT, RPT)], out_hbm.at[c, pl.ds(s * RPT, RPT)])


@functools.partial(
    pl.kernel,
    out_type=jax.ShapeDtypeStruct((NC, N_PAD, D), jnp.float32),
    mesh=_MESH,
    scratch_types=[
        pltpu.VMEM((PH, CHUNK), jnp.int32),
        pltpu.VMEM((PH, CHUNK), jnp.int32),
        pltpu.VMEM((2, CHUNK, D), jnp.float32),
        pltpu.VMEM((ZR, D), jnp.float32),
        pltpu.VMEM_SHARED((N_PAD, D), jnp.float32),
        pltpu.SemaphoreType.DMA,
        pltpu.SemaphoreType.DMA,
    ],
)
def _hop_kernel(g_hbm, src_hbm, dst_hbm, out_hbm, sidx, didx, rows, zbuf,
                acc, gs0, gs1):
    gsem = (gs0, gs1)
    c = lax.axis_index("c")
    s = lax.axis_index("s")
    wid = s * NC + c

    def fill_zeros(k, _):
        zbuf[k // 8, pl.ds((k % 8) * 16, 16)] = jnp.zeros((16,), jnp.float32)
        return _

    lax.fori_loop(0, ZR * (D // 16), fill_zeros, None)

    def zero_acc(k, _):
        pltpu.sync_copy(zbuf, acc.at[pl.ds(s * RPT + k * ZR, ZR)])
        return _

    lax.fori_loop(0, RPT // ZR, zero_acc, None)
    plsc.subcore_barrier()

    for p in range(CPT // PH):
        pltpu.sync_copy(src_hbm.at[wid, pl.ds(p * PH, PH)], sidx)
        pltpu.sync_copy(dst_hbm.at[wid, pl.ds(p * PH, PH)], didx)
        pltpu.async_copy(g_hbm.at[sidx.at[0]], rows.at[0], gs0)
        pltpu.async_copy(g_hbm.at[sidx.at[1]], rows.at[1], gs1)

        def pair(j, _):
            for bb in range(2):
                k = j * 2 + bb
                pltpu.make_async_copy(
                    g_hbm.at[sidx.at[k]], rows.at[bb], gsem[bb]
                ).wait()
                pltpu.sync_copy(rows.at[bb], acc.at[didx.at[k]], add=True)
                pltpu.async_copy(g_hbm.at[sidx.at[k + 2]], rows.at[bb], gsem[bb])
            return _

        lax.fori_loop(0, PH // 2 - 1, pair, None)

        for k in (PH - 2, PH - 1):
            bb = k % 2
            pltpu.make_async_copy(
                g_hbm.at[sidx.at[k]], rows.at[bb], gsem[bb]
            ).wait()
            pltpu.sync_copy(rows.at[bb], acc.at[didx.at[k]], add=True)

    plsc.subcore_barrier()

    def flush(k, _):
        sl = pl.ds(s * RPT + k * ZR, ZR)
        pltpu.sync_copy(acc.at[sl], out_hbm.at[c, sl])
        return _

    lax.fori_loop(0, RPT // ZR, flush, None)


_ROWS = 1000
_GRID = N // _ROWS


def _scale0_body(deg_ref, x_ref, o_ref):
    dsum = deg_ref[0] + deg_ref[1] + 1.0
    o_ref[...] = lax.rsqrt(dsum) * x_ref[...]


def _scale_mid_body(deg_ref, parts_ref, g_ref, o_ref):
    dsum = deg_ref[0] + deg_ref[1] + 1.0
    o_ref[...] = (parts_ref[0] + parts_ref[1] + g_ref[...]) / dsum


def _final_body(deg_ref, parts_ref, g_ref, w_ref, o_ref):
    dsum = deg_ref[0] + deg_ref[1] + 1.0
    h = lax.rsqrt(dsum) * (parts_ref[0] + parts_ref[1] + g_ref[...])
    o_ref[...] = jnp.dot(h, w_ref[...], preferred_element_type=jnp.float32)


_deg_spec = pl.BlockSpec((NC, _ROWS, 1), lambda i: (0, i, 0))
_row_spec = pl.BlockSpec((_ROWS, D), lambda i: (i, 0))
_parts_spec = pl.BlockSpec((NC, _ROWS, D), lambda i: (0, i, 0))
_w_spec = pl.BlockSpec((D, D), lambda i: (0, 0))
_out_shape = jax.ShapeDtypeStruct((N, D), jnp.float32)

_scale0 = pl.pallas_call(
    _scale0_body, grid=(_GRID,),
    in_specs=[_deg_spec, _row_spec], out_specs=_row_spec, out_shape=_out_shape,
)
_scale_mid = pl.pallas_call(
    _scale_mid_body, grid=(_GRID,),
    in_specs=[_deg_spec, _parts_spec, _row_spec],
    out_specs=_row_spec, out_shape=_out_shape,
)
_final = pl.pallas_call(
    _final_body, grid=(_GRID,),
    in_specs=[_deg_spec, _parts_spec, _row_spec, _w_spec],
    out_specs=_row_spec, out_shape=_out_shape,
)


@jax.jit
def kernel(x, edge_index, W):
    src = edge_index[0].astype(jnp.int32)
    dst = edge_index[1].astype(jnp.int32)
    pad = E_PAD - E
    ar = jnp.arange(pad, dtype=jnp.int32)
    src_pad = ar % N
    dst_pad = N + 1 + ar % (N_PAD - N - 1)
    src3 = jnp.concatenate([src, src_pad]).reshape(NW, CPT, CHUNK)
    dst3 = jnp.concatenate([dst, dst_pad]).reshape(NW, CPT, CHUNK)

    deg_parts = _deg_kernel(dst3).reshape(NC, N_PAD, 1)
    g0 = _scale0(deg_parts, x)
    p1 = _hop_kernel(g0, src3, dst3)
    g1 = _scale_mid(deg_parts, p1, g0)
    p2 = _hop_kernel(g1, src3, dst3)
    return _final(deg_parts, p2, g1, W)


# speedup vs baseline: 4.4825x; 1.0334x over previous
import functools

import jax
import jax.numpy as jnp
from jax import lax
from jax.experimental import pallas as pl
from jax.experimental.pallas import tpu as pltpu
from jax.experimental.pallas import tpu_sc as plsc

N = 10000
E = 320000
D = 128

NC = 2
NS = 16
NW = NC * NS
CHUNK = 128
CPT = 80
PH = 40
EPT = CPT * CHUNK
E_PAD = NW * EPT
N_PAD = 10240
RPT = N_PAD // NS
ZR = 40
ER = E // CHUNK
REM = 16
AUXR = NW * CPT - (NW - 1) * CPT - REM + (NW * CPT - ER)
AUX = 64

_MESH = plsc.VectorSubcoreMesh(
    core_axis_name="c", subcore_axis_name="s", num_cores=NC, num_subcores=NS
)


@functools.partial(
    pl.kernel,
    out_type=jax.ShapeDtypeStruct((NC, N_PAD), jnp.float32),
    mesh=_MESH,
    scratch_types=[
        pltpu.VMEM((CPT, CHUNK), jnp.int32),
        pltpu.VMEM((CHUNK,), jnp.float32),
        pltpu.VMEM((RPT,), jnp.float32),
        pltpu.VMEM_SHARED((N_PAD,), jnp.float32),
    ],
)
def _deg_kernel(e2_hbm, aux_hbm, out_hbm, didx, ones, zrow, deg):
    c = lax.axis_index("c")
    s = lax.axis_index("s")
    wid = s * NC + c

    def fill_ones(k, _):
        ones[pl.ds(k * 16, 16)] = jnp.full((16,), 1.0, jnp.float32)
        return _

    lax.fori_loop(0, CHUNK // 16, fill_ones, None)

    def fill_zeros(k, _):
        zrow[pl.ds(k * 16, 16)] = jnp.zeros((16,), jnp.float32)
        return _

    lax.fori_loop(0, RPT // 16, fill_zeros, None)

    @pl.when(wid != NW - 1)
    def _():
        pltpu.sync_copy(e2_hbm.at[1, pl.ds(wid * CPT, CPT)], didx)

    @pl.when(wid == NW - 1)
    def _():
        pltpu.sync_copy(e2_hbm.at[1, pl.ds((NW - 1) * CPT, REM)],
                        didx.at[pl.ds(0, REM)])
        pltpu.sync_copy(aux_hbm.at[1], didx.at[pl.ds(REM, AUX)])

    pltpu.sync_copy(zrow, deg.at[pl.ds(s * RPT, RPT)])
    plsc.subcore_barrier()

    def chunk(ci, _):
        pltpu.sync_copy(ones, deg.at[didx.at[ci]], add=True)
        return _

    lax.fori_loop(0, CPT, chunk, None)
    plsc.subcore_barrier()
    pltpu.sync_copy(deg.at[pl.ds(s * RPT, RPT)], out_hbm.at[c, pl.ds(s * RPT, RPT)])


@functools.partial(
    pl.kernel,
    out_type=jax.ShapeDtypeStruct((NC, N_PAD, D), jnp.float32),
    mesh=_MESH,
    scratch_types=[
        pltpu.VMEM((PH, CHUNK), jnp.int32),
        pltpu.VMEM((PH, CHUNK), jnp.int32),
        pltpu.VMEM((2, CHUNK, D), jnp.float32),
        pltpu.VMEM((ZR, D), jnp.float32),
        pltpu.VMEM_SHARED((N_PAD, D), jnp.float32),
        pltpu.SemaphoreType.DMA,
        pltpu.SemaphoreType.DMA,
    ],
)
def _hop_kernel(g_hbm, e2_hbm, aux_hbm, out_hbm, sidx, didx, rows,
                zbuf, acc, gs0, gs1):
    gsem = (gs0, gs1)
    c = lax.axis_index("c")
    s = lax.axis_index("s")
    wid = s * NC + c

    def fill_zeros(k, _):
        zbuf[k // 8, pl.ds((k % 8) * 16, 16)] = jnp.zeros((16,), jnp.float32)
        return _

    lax.fori_loop(0, ZR * (D // 16), fill_zeros, None)

    def zero_acc(k, _):
        pltpu.sync_copy(zbuf, acc.at[pl.ds(s * RPT + k * ZR, ZR)])
        return _

    lax.fori_loop(0, RPT // ZR, zero_acc, None)
    plsc.subcore_barrier()

    for p in range(CPT // PH):
        @pl.when(wid != NW - 1)
        def _():
            base = wid * CPT + p * PH
            pltpu.sync_copy(e2_hbm.at[0, pl.ds(base, PH)], sidx)
            pltpu.sync_copy(e2_hbm.at[1, pl.ds(base, PH)], didx)

        @pl.when(wid == NW - 1)
        def _():
            if p == 0:
                pltpu.sync_copy(e2_hbm.at[0, pl.ds((NW - 1) * CPT, REM)],
                                sidx.at[pl.ds(0, REM)])
                pltpu.sync_copy(e2_hbm.at[1, pl.ds((NW - 1) * CPT, REM)],
                                didx.at[pl.ds(0, REM)])
                pltpu.sync_copy(aux_hbm.at[0, pl.ds(0, PH - REM)],
                                sidx.at[pl.ds(REM, PH - REM)])
                pltpu.sync_copy(aux_hbm.at[1, pl.ds(0, PH - REM)],
                                didx.at[pl.ds(REM, PH - REM)])
            else:
                pltpu.sync_copy(aux_hbm.at[0, pl.ds(PH - REM, PH)], sidx)
                pltpu.sync_copy(aux_hbm.at[1, pl.ds(PH - REM, PH)], didx)
        pltpu.async_copy(g_hbm.at[sidx.at[0]], rows.at[0], gs0)
        pltpu.async_copy(g_hbm.at[sidx.at[1]], rows.at[1], gs1)

        def pair(j, _):
            for bb in range(2):
                k = j * 2 + bb
                pltpu.make_async_copy(
                    g_hbm.at[sidx.at[k]], rows.at[bb], gsem[bb]
                ).wait()
                pltpu.sync_copy(rows.at[bb], acc.at[didx.at[k]], add=True)
                pltpu.async_copy(g_hbm.at[sidx.at[k + 2]], rows.at[bb], gsem[bb])
            return _

        lax.fori_loop(0, PH // 2 - 1, pair, None)

        for k in (PH - 2, PH - 1):
            bb = k % 2
            pltpu.make_async_copy(
                g_hbm.at[sidx.at[k]], rows.at[bb], gsem[bb]
            ).wait()
            pltpu.sync_copy(rows.at[bb], acc.at[didx.at[k]], add=True)

    plsc.subcore_barrier()

    def flush(k, _):
        sl = pl.ds(s * RPT + k * ZR, ZR)
        pltpu.sync_copy(acc.at[sl], out_hbm.at[c, sl])
        return _

    lax.fori_loop(0, RPT // ZR, flush, None)


_ROWS = 1000
_GRID = N // _ROWS


def _scale0_body(deg_ref, x_ref, o_ref):
    dsum = deg_ref[0] + deg_ref[1] + 1.0
    o_ref[...] = lax.rsqrt(dsum) * x_ref[...]


def _scale_mid_body(deg_ref, parts_ref, g_ref, o_ref):
    dsum = deg_ref[0] + deg_ref[1] + 1.0
    o_ref[...] = (parts_ref[0] + parts_ref[1] + g_ref[...]) / dsum


def _final_body(deg_ref, parts_ref, g_ref, w_ref, o_ref):
    dsum = deg_ref[0] + deg_ref[1] + 1.0
    h = lax.rsqrt(dsum) * (parts_ref[0] + parts_ref[1] + g_ref[...])
    o_ref[...] = jnp.dot(h, w_ref[...], preferred_element_type=jnp.float32)


_deg_spec = pl.BlockSpec((NC, _ROWS, 1), lambda i: (0, i, 0))
_row_spec = pl.BlockSpec((_ROWS, D), lambda i: (i, 0))
_parts_spec = pl.BlockSpec((NC, _ROWS, D), lambda i: (0, i, 0))
_w_spec = pl.BlockSpec((D, D), lambda i: (0, 0))
_out_shape = jax.ShapeDtypeStruct((N, D), jnp.float32)

_scale0 = pl.pallas_call(
    _scale0_body, grid=(_GRID,),
    in_specs=[_deg_spec, _row_spec], out_specs=_row_spec, out_shape=_out_shape,
)
_scale_mid = pl.pallas_call(
    _scale_mid_body, grid=(_GRID,),
    in_specs=[_deg_spec, _parts_spec, _row_spec],
    out_specs=_row_spec, out_shape=_out_shape,
)
_final = pl.pallas_call(
    _final_body, grid=(_GRID,),
    in_specs=[_deg_spec, _parts_spec, _row_spec, _w_spec],
    out_specs=_row_spec, out_shape=_out_shape,
)


@jax.jit
def kernel(x, edge_index, W):
    e2 = edge_index.astype(jnp.int32).reshape(2, ER, CHUNK)
    npad = AUX - 4
    ar = jnp.arange(npad * CHUNK, dtype=jnp.int32)
    psrc = (ar % N).reshape(1, npad, CHUNK)
    pdst = (N + 1 + ar % (N_PAD - N - 1)).reshape(1, npad, CHUNK)
    aux = jnp.concatenate(
        [e2[:, (NW - 1) * CPT + REM:], jnp.concatenate([psrc, pdst], 0)], axis=1
    )

    deg_parts = _deg_kernel(e2, aux).reshape(NC, N_PAD, 1)
    g0 = _scale0(deg_parts, x)
    p1 = _hop_kernel(g0, e2, aux)
    g1 = _scale_mid(deg_parts, p1, g0)
    p2 = _hop_kernel(g1, e2, aux)
    return _final(deg_parts, p2, g1, W)


# speedup vs baseline: 4.4854x; 1.0006x over previous
import functools

import jax
import jax.numpy as jnp
from jax import lax
from jax.experimental import pallas as pl
from jax.experimental.pallas import tpu as pltpu
from jax.experimental.pallas import tpu_sc as plsc

N = 10000
E = 320000
D = 128

NC = 2
NS = 16
NW = NC * NS
CHUNK = 128
CPT = 80
PH = 40
EPT = CPT * CHUNK
E_PAD = NW * EPT
N_PAD = 10240
RPT = N_PAD // NS
ZR = 40
ER = E // CHUNK
AUX = 80

_MESH = plsc.VectorSubcoreMesh(
    core_axis_name="c", subcore_axis_name="s", num_cores=NC, num_subcores=NS
)


@functools.partial(
    pl.kernel,
    out_type=jax.ShapeDtypeStruct((NC, N_PAD), jnp.float32),
    mesh=_MESH,
    scratch_types=[
        pltpu.VMEM((CPT, CHUNK), jnp.int32),
        pltpu.VMEM((CHUNK,), jnp.float32),
        pltpu.VMEM((RPT,), jnp.float32),
        pltpu.VMEM_SHARED((N_PAD,), jnp.float32),
    ],
)
def _deg_kernel(e2_hbm, aux_hbm, out_hbm, didx, ones, zrow, deg):
    c = lax.axis_index("c")
    s = lax.axis_index("s")
    wid = s * NC + c

    def fill_ones(k, _):
        ones[pl.ds(k * 16, 16)] = jnp.full((16,), 1.0, jnp.float32)
        return _

    lax.fori_loop(0, CHUNK // 16, fill_ones, None)

    def fill_zeros(k, _):
        zrow[pl.ds(k * 16, 16)] = jnp.zeros((16,), jnp.float32)
        return _

    lax.fori_loop(0, RPT // 16, fill_zeros, None)

    @pl.when(wid != NW - 1)
    def _():
        pltpu.sync_copy(e2_hbm.at[1, pl.ds(wid * CPT, CPT)], didx)

    @pl.when(wid == NW - 1)
    def _():
        pltpu.sync_copy(aux_hbm.at[1], didx)

    pltpu.sync_copy(zrow, deg.at[pl.ds(s * RPT, RPT)])
    plsc.subcore_barrier()

    def chunk(ci, _):
        pltpu.sync_copy(ones, deg.at[didx.at[ci]], add=True)
        return _

    lax.fori_loop(0, CPT, chunk, None)
    plsc.subcore_barrier()
    pltpu.sync_copy(deg.at[pl.ds(s * RPT, RPT)], out_hbm.at[c, pl.ds(s * RPT, RPT)])


@functools.partial(
    pl.kernel,
    out_type=jax.ShapeDtypeStruct((NC, N_PAD, D), jnp.float32),
    mesh=_MESH,
    scratch_types=[
        pltpu.VMEM((PH, CHUNK), jnp.int32),
        pltpu.VMEM((PH, CHUNK), jnp.int32),
        pltpu.VMEM((2, CHUNK, D), jnp.float32),
        pltpu.VMEM((ZR, D), jnp.float32),
        pltpu.VMEM_SHARED((N_PAD, D), jnp.float32),
        pltpu.SemaphoreType.DMA,
        pltpu.SemaphoreType.DMA,
    ],
)
def _hop_kernel(g_hbm, e2_hbm, aux_hbm, out_hbm, sidx, didx, rows,
                zbuf, acc, gs0, gs1):
    gsem = (gs0, gs1)
    c = lax.axis_index("c")
    s = lax.axis_index("s")
    wid = s * NC + c

    def fill_zeros(k, _):
        zbuf[k // 8, pl.ds((k % 8) * 16, 16)] = jnp.zeros((16,), jnp.float32)
        return _

    lax.fori_loop(0, ZR * (D // 16), fill_zeros, None)

    def zero_acc(k, _):
        pltpu.sync_copy(zbuf, acc.at[pl.ds(s * RPT + k * ZR, ZR)])
        return _

    lax.fori_loop(0, RPT // ZR, zero_acc, None)
    plsc.subcore_barrier()

    for p in range(CPT // PH):
        @pl.when(wid != NW - 1)
        def _():
            base = wid * CPT + p * PH
            pltpu.sync_copy(e2_hbm.at[0, pl.ds(base, PH)], sidx)
            pltpu.sync_copy(e2_hbm.at[1, pl.ds(base, PH)], didx)

        @pl.when(wid == NW - 1)
        def _():
            pltpu.sync_copy(aux_hbm.at[0, pl.ds(p * PH, PH)], sidx)
            pltpu.sync_copy(aux_hbm.at[1, pl.ds(p * PH, PH)], didx)
        pltpu.async_copy(g_hbm.at[sidx.at[0]], rows.at[0], gs0)
        pltpu.async_copy(g_hbm.at[sidx.at[1]], rows.at[1], gs1)

        def pair(j, _):
            for bb in range(2):
                k = j * 2 + bb
                pltpu.make_async_copy(
                    g_hbm.at[sidx.at[k]], rows.at[bb], gsem[bb]
                ).wait()
                pltpu.sync_copy(rows.at[bb], acc.at[didx.at[k]], add=True)
                pltpu.async_copy(g_hbm.at[sidx.at[k + 2]], rows.at[bb], gsem[bb])
            return _

        lax.fori_loop(0, PH // 2 - 1, pair, None)

        for k in (PH - 2, PH - 1):
            bb = k % 2
            pltpu.make_async_copy(
                g_hbm.at[sidx.at[k]], rows.at[bb], gsem[bb]
            ).wait()
            pltpu.sync_copy(rows.at[bb], acc.at[didx.at[k]], add=True)

    plsc.subcore_barrier()

    def flush(k, _):
        sl = pl.ds(s * RPT + k * ZR, ZR)
        pltpu.sync_copy(acc.at[sl], out_hbm.at[c, sl])
        return _

    lax.fori_loop(0, RPT // ZR, flush, None)


_ROWS = 1000
_GRID = N // _ROWS


def _scale0_body(deg_ref, x_ref, o_ref):
    dsum = deg_ref[0] + deg_ref[1] + 1.0
    o_ref[...] = lax.rsqrt(dsum) * x_ref[...]


def _scale_mid_body(deg_ref, parts_ref, g_ref, o_ref):
    dsum = deg_ref[0] + deg_ref[1] + 1.0
    o_ref[...] = (parts_ref[0] + parts_ref[1] + g_ref[...]) / dsum


def _final_body(deg_ref, parts_ref, g_ref, w_ref, o_ref):
    dsum = deg_ref[0] + deg_ref[1] + 1.0
    h = lax.rsqrt(dsum) * (parts_ref[0] + parts_ref[1] + g_ref[...])
    o_ref[...] = jnp.dot(h, w_ref[...], preferred_element_type=jnp.float32)


_deg_spec = pl.BlockSpec((NC, _ROWS, 1), lambda i: (0, i, 0))
_row_spec = pl.BlockSpec((_ROWS, D), lambda i: (i, 0))
_parts_spec = pl.BlockSpec((NC, _ROWS, D), lambda i: (0, i, 0))
_w_spec = pl.BlockSpec((D, D), lambda i: (0, 0))
_out_shape = jax.ShapeDtypeStruct((N, D), jnp.float32)

_scale0 = pl.pallas_call(
    _scale0_body, grid=(_GRID,),
    in_specs=[_deg_spec, _row_spec], out_specs=_row_spec, out_shape=_out_shape,
)
_scale_mid = pl.pallas_call(
    _scale_mid_body, grid=(_GRID,),
    in_specs=[_deg_spec, _parts_spec, _row_spec],
    out_specs=_row_spec, out_shape=_out_shape,
)
_final = pl.pallas_call(
    _final_body, grid=(_GRID,),
    in_specs=[_deg_spec, _parts_spec, _row_spec, _w_spec],
    out_specs=_row_spec, out_shape=_out_shape,
)


@jax.jit
def kernel(x, edge_index, W):
    e2 = edge_index.astype(jnp.int32).reshape(2, ER, CHUNK)
    nreal = ER - (NW - 1) * CPT
    npad = AUX - nreal
    ar = jnp.arange(npad * CHUNK, dtype=jnp.int32)
    psrc = (ar % N).reshape(1, npad, CHUNK)
    pdst = (N + 1 + ar % (N_PAD - N - 1)).reshape(1, npad, CHUNK)
    aux = jnp.concatenate(
        [e2[:, (NW - 1) * CPT:], jnp.concatenate([psrc, pdst], 0)], axis=1
    )

    deg_parts = _deg_kernel(e2, aux).reshape(NC, N_PAD, 1)
    g0 = _scale0(deg_parts, x)
    p1 = _hop_kernel(g0, e2, aux)
    g1 = _scale_mid(deg_parts, p1, g0)
    p2 = _hop_kernel(g1, e2, aux)
    return _final(deg_parts, p2, g1, W)
